# Initial kernel scaffold; baseline (speedup 1.0000x reference)
import jax
import jax.numpy as jnp
from jax.experimental import pallas as pl


def kernel(input_feats, sampling_grid, codebook):
    raise NotImplementedError("write your pallas kernel here")


# speedup vs baseline: 6.9395x; 6.9395x over previous
import jax
import jax.numpy as jnp
from jax import lax
from jax.experimental import pallas as pl
from jax.experimental.pallas import tpu as pltpu
from jax.experimental.pallas import tpu_sc as plsc

_B, _C, _D, _H, _W = 2, 256, 16, 32, 32
_GW = 2048
_K = 1024
_NCORN = 8
_NROWS = _B * _NCORN * _GW
_GWIN = 128
_TG = 128


def _axis_idx(c, n):
    half = (n - 1.0) / 2.0
    cg = (c - half) / half
    cg = jnp.clip(cg, -1.0, 1.0)
    i = ((cg + 1.0) * n - 1.0) / 2.0
    ri = jnp.round(i).astype(jnp.int32)
    valid = (ri >= 0) & (ri < n)
    return jnp.clip(ri, 0, int(n) - 1), valid


def _idx_kernel(g_ref, idx_ref, w_ref, m_ref):
    xs = g_ref[0]
    ys = g_ref[1]
    zs = g_ref[2]
    x = jnp.clip(xs, -1.0, 1.0)
    x = (x + 1.0) / 2.0
    x = x * (_W - 1.0)
    y = jnp.clip(ys, -1.0, 1.0)
    y = (y + 1.0) / 2.0
    y = y * (_H - 1.0)
    z = jnp.clip(zs, -1.0, 1.0)
    z = (z + 1.0) / 2.0
    z = z * (_D - 1.0)
    x0 = jnp.floor(x)
    y0 = jnp.floor(y)
    z0 = jnp.floor(z)
    u = x - x0
    v = y - y0
    w = z - z0
    x1 = x0 + 1.0
    y1 = y0 + 1.0
    z1 = z0 + 1.0
    b = lax.broadcasted_iota(jnp.int32, (_B, _GW), 0)
    for j in range(_NCORN):
        dx, dy, dz = (j >> 2) & 1, (j >> 1) & 1, j & 1
        cx = x1 if dx else x0
        cy = y1 if dy else y0
        cz = z1 if dz else z0
        ix, mx = _axis_idx(cx, _W)
        iy, my = _axis_idx(cy, _H)
        iz = jnp.clip(cz.astype(jnp.int32), 0, _D - 1)
        idx_ref[j] = b * (_D * _H * _W) + iz * (_H * _W) + iy * _W + ix
        m_ref[j] = (mx & my).astype(jnp.float32)
        fx = u if dx else (1.0 - u)
        fy = v if dy else (1.0 - v)
        fz = w if dz else (1.0 - w)
        w_ref[j] = fx * fy * fz


def _compute_indices(g3):
    return pl.pallas_call(
        _idx_kernel,
        out_shape=(
            jax.ShapeDtypeStruct((_NCORN, _B, _GW), jnp.int32),
            jax.ShapeDtypeStruct((_NCORN, _B, _GW), jnp.float32),
            jax.ShapeDtypeStruct((_NCORN, _B, _GW), jnp.float32),
        ),
    )(g3)


def _sc_gather(feats_rows, idx_flat):
    mesh = plsc.VectorSubcoreMesh(core_axis_name="core",
                                  subcore_axis_name="subcore")

    @pl.kernel(out_type=jax.ShapeDtypeStruct((_NROWS, _C), jnp.float32),
               mesh=mesh)
    def gather_kernel(data_hbm, i_hbm, o_hbm):
        def body(i_vmem, o_vmem):
            pltpu.sync_copy(data_hbm.at[i_vmem.at[0]], o_vmem)

        pltpu.emit_pipeline(
            body,
            grid=(_NROWS // _GWIN,),
            in_specs=[pl.BlockSpec((1, _GWIN), lambda i: (0, i))],
            out_specs=[pl.BlockSpec((_GWIN, _C), lambda i: (i, 0))],
            core_axis_name=("core", "subcore"),
            dimension_semantics=(pltpu.PARALLEL,),
        )(i_hbm, o_hbm)

    return gather_kernel(feats_rows, idx_flat)


def _vq_kernel(x_ref, w_ref, m_ref, cb_ref, o_ref):
    rows = _NCORN * _TG
    cb = cb_ref[...]
    x3 = x_ref[...].reshape(_NCORN, _TG, _C)
    mt = m_ref[...].reshape(_NCORN, _TG, 1)
    wt = w_ref[...].reshape(_NCORN, _TG, 1)
    x2 = (x3 * mt).reshape(rows, _C)
    xx = jnp.sum(x2 * x2, axis=-1, keepdims=True)
    dot = lax.dot_general(x2, cb, (((1,), (1,)), ((), ())),
                          preferred_element_type=jnp.float32,
                          precision=lax.Precision.DEFAULT)
    cc = jnp.sum(cb * cb, axis=-1)
    d = xx - 2.0 * dot + cc
    dmin = jnp.min(d, axis=-1, keepdims=True)
    kk = lax.broadcasted_iota(jnp.int32, (rows, _K), 1)
    cand = jnp.where(d == dmin, kk, _K)
    sel = jnp.min(cand, axis=-1, keepdims=True)
    onehot = (sel == kk).astype(jnp.float32)
    q = lax.dot_general(onehot, cb, (((1,), (0,)), ((), ())),
                        preferred_element_type=jnp.float32,
                        precision=lax.Precision.HIGHEST)
    q3 = q.reshape(_NCORN, _TG, _C)
    acc = wt[0] * q3[0]
    for j in range(1, _NCORN):
        acc = acc + wt[j] * q3[j]
    o_ref[...] = acc.reshape(1, _TG, _C)


def _vq_blend(x4, w8, m8, codebook):
    return pl.pallas_call(
        _vq_kernel,
        grid=(_B, _GW // _TG),
        in_specs=[
            pl.BlockSpec((_NCORN, 1, _TG, _C), lambda b, t: (0, b, t, 0)),
            pl.BlockSpec((_NCORN, 1, 1, _TG), lambda b, t: (0, b, 0, t)),
            pl.BlockSpec((_NCORN, 1, 1, _TG), lambda b, t: (0, b, 0, t)),
            pl.BlockSpec((_K, _C), lambda b, t: (0, 0)),
        ],
        out_specs=pl.BlockSpec((1, _TG, _C), lambda b, t: (b, t, 0)),
        out_shape=jax.ShapeDtypeStruct((_B, _GW, _C), jnp.float32),
    )(x4, w8.reshape(_NCORN, _B, 1, _GW), m8.reshape(_NCORN, _B, 1, _GW),
      codebook)


def kernel(input_feats, sampling_grid, codebook):
    g3 = jnp.transpose(sampling_grid.reshape(_B, _GW, 3), (2, 0, 1))
    idx, w8, m8 = _compute_indices(g3)
    feats_rows = jnp.transpose(input_feats, (0, 2, 3, 4, 1))
    feats_rows = feats_rows.reshape(_B * _D * _H * _W, _C)
    xg = _sc_gather(feats_rows, idx.reshape(1, _NROWS))
    x4 = xg.reshape(_NCORN, _B, _GW, _C)
    out = _vq_blend(x4, w8, m8, codebook)
    return jnp.transpose(out, (0, 2, 1))[:, :, None, :]


# speedup vs baseline: 10.5104x; 1.5146x over previous
import jax
import jax.numpy as jnp
from jax import lax
from jax.experimental import pallas as pl
from jax.experimental.pallas import tpu as pltpu
from jax.experimental.pallas import tpu_sc as plsc

_B, _C, _D, _H, _W = 2, 256, 16, 32, 32
_GW = 2048
_K = 1024
_NCORN = 8
_NROWS = _B * _NCORN * _GW
_GWIN = 128
_TG = 128


def _axis_idx(c, n):
    half = (n - 1.0) / 2.0
    cg = (c - half) / half
    cg = jnp.clip(cg, -1.0, 1.0)
    i = ((cg + 1.0) * n - 1.0) / 2.0
    ri = jnp.round(i).astype(jnp.int32)
    valid = (ri >= 0) & (ri < n)
    return jnp.clip(ri, 0, int(n) - 1), valid


def _idx_kernel(g_ref, idx_ref, w_ref, m_ref):
    xs = g_ref[0]
    ys = g_ref[1]
    zs = g_ref[2]
    x = jnp.clip(xs, -1.0, 1.0)
    x = (x + 1.0) / 2.0
    x = x * (_W - 1.0)
    y = jnp.clip(ys, -1.0, 1.0)
    y = (y + 1.0) / 2.0
    y = y * (_H - 1.0)
    z = jnp.clip(zs, -1.0, 1.0)
    z = (z + 1.0) / 2.0
    z = z * (_D - 1.0)
    x0 = jnp.floor(x)
    y0 = jnp.floor(y)
    z0 = jnp.floor(z)
    u = x - x0
    v = y - y0
    w = z - z0
    x1 = x0 + 1.0
    y1 = y0 + 1.0
    z1 = z0 + 1.0
    b = lax.broadcasted_iota(jnp.int32, (_B, _GW), 0)
    for j in range(_NCORN):
        dx, dy, dz = (j >> 2) & 1, (j >> 1) & 1, j & 1
        cx = x1 if dx else x0
        cy = y1 if dy else y0
        cz = z1 if dz else z0
        ix, mx = _axis_idx(cx, _W)
        iy, my = _axis_idx(cy, _H)
        iz = jnp.clip(cz.astype(jnp.int32), 0, _D - 1)
        idx_ref[j] = b * (_D * _H * _W) + iz * (_H * _W) + iy * _W + ix
        m_ref[j] = (mx & my).astype(jnp.float32)
        fx = u if dx else (1.0 - u)
        fy = v if dy else (1.0 - v)
        fz = w if dz else (1.0 - w)
        w_ref[j] = fx * fy * fz


def _compute_indices(g3):
    return pl.pallas_call(
        _idx_kernel,
        out_shape=(
            jax.ShapeDtypeStruct((_NCORN, _B, _GW), jnp.int32),
            jax.ShapeDtypeStruct((_NCORN, _B, _GW), jnp.float32),
            jax.ShapeDtypeStruct((_NCORN, _B, _GW), jnp.float32),
        ),
    )(g3)


def _sc_gather(feats_rows, idx_flat):
    mesh = plsc.VectorSubcoreMesh(core_axis_name="core",
                                  subcore_axis_name="subcore")

    @pl.kernel(out_type=jax.ShapeDtypeStruct((_NROWS, _C), jnp.float32),
               mesh=mesh)
    def gather_kernel(data_hbm, i_hbm, o_hbm):
        def body(i_vmem, o_vmem):
            pltpu.sync_copy(data_hbm.at[i_vmem.at[0]], o_vmem)

        pltpu.emit_pipeline(
            body,
            grid=(_NROWS // _GWIN,),
            in_specs=[pl.BlockSpec((1, _GWIN), lambda i: (0, i))],
            out_specs=[pl.BlockSpec((_GWIN, _C), lambda i: (i, 0))],
            core_axis_name=("core", "subcore"),
            dimension_semantics=(pltpu.PARALLEL,),
        )(i_hbm, o_hbm)

    return gather_kernel(feats_rows, idx_flat)


def _vq_kernel(x_ref, w_ref, m_ref, cb_ref, o_ref):
    rows = _NCORN * _TG
    cb = cb_ref[...]
    x3 = x_ref[...].reshape(_NCORN, _TG, _C)
    mt = m_ref[...].reshape(_NCORN, _TG, 1)
    wt = w_ref[...].reshape(_NCORN, _TG, 1)
    x2 = (x3 * mt).reshape(rows, _C)
    xx = jnp.sum(x2 * x2, axis=-1, keepdims=True)
    dot = lax.dot_general(x2, cb, (((1,), (1,)), ((), ())),
                          preferred_element_type=jnp.float32,
                          precision=lax.Precision.DEFAULT)
    cc = jnp.sum(cb * cb, axis=-1)
    d = xx - 2.0 * dot + cc
    dmin = jnp.min(d, axis=-1, keepdims=True)
    kk = lax.broadcasted_iota(jnp.int32, (rows, _K), 1)
    cand = jnp.where(d == dmin, kk, _K)
    sel = jnp.min(cand, axis=-1, keepdims=True)
    sel3 = sel.reshape(_NCORN, _TG, 1)
    kk2 = lax.broadcasted_iota(jnp.int32, (_TG, _K), 1)
    msel = jnp.where(sel3[0] == kk2, wt[0], 0.0)
    for j in range(1, _NCORN):
        msel = msel + jnp.where(sel3[j] == kk2, wt[j], 0.0)
    q = lax.dot_general(msel, cb, (((1,), (0,)), ((), ())),
                        preferred_element_type=jnp.float32,
                        precision=lax.Precision.HIGHEST)
    o_ref[...] = q.reshape(1, _TG, _C)


def _vq_blend(x4, w8, m8, codebook):
    return pl.pallas_call(
        _vq_kernel,
        grid=(_B, _GW // _TG),
        in_specs=[
            pl.BlockSpec((_NCORN, 1, _TG, _C), lambda b, t: (0, b, t, 0)),
            pl.BlockSpec((_NCORN, 1, 1, _TG), lambda b, t: (0, b, 0, t)),
            pl.BlockSpec((_NCORN, 1, 1, _TG), lambda b, t: (0, b, 0, t)),
            pl.BlockSpec((_K, _C), lambda b, t: (0, 0)),
        ],
        out_specs=pl.BlockSpec((1, _TG, _C), lambda b, t: (b, t, 0)),
        out_shape=jax.ShapeDtypeStruct((_B, _GW, _C), jnp.float32),
    )(x4, w8.reshape(_NCORN, _B, 1, _GW), m8.reshape(_NCORN, _B, 1, _GW),
      codebook)


def kernel(input_feats, sampling_grid, codebook):
    g3 = jnp.transpose(sampling_grid.reshape(_B, _GW, 3), (2, 0, 1))
    idx, w8, m8 = _compute_indices(g3)
    feats_rows = jnp.transpose(input_feats, (0, 2, 3, 4, 1))
    feats_rows = feats_rows.reshape(_B * _D * _H * _W, _C)
    xg = _sc_gather(feats_rows, idx.reshape(1, _NROWS))
    x4 = xg.reshape(_NCORN, _B, _GW, _C)
    out = _vq_blend(x4, w8, m8, codebook)
    return jnp.transpose(out, (0, 2, 1))[:, :, None, :]


# speedup vs baseline: 10.8817x; 1.0353x over previous
import jax
import jax.numpy as jnp
from jax import lax
from jax.experimental import pallas as pl
from jax.experimental.pallas import tpu as pltpu
from jax.experimental.pallas import tpu_sc as plsc

_B, _C, _D, _H, _W = 2, 256, 16, 32, 32
_GW = 2048
_K = 1024
_NCORN = 8
_NROWS = _B * _NCORN * _GW
_GWIN = 128
_TG = 256


def _axis_idx(c, n):
    half = (n - 1.0) / 2.0
    cg = (c - half) / half
    cg = jnp.clip(cg, -1.0, 1.0)
    i = ((cg + 1.0) * n - 1.0) / 2.0
    ri = jnp.round(i).astype(jnp.int32)
    valid = (ri >= 0) & (ri < n)
    return jnp.clip(ri, 0, int(n) - 1), valid


def _idx_kernel(g_ref, idx_ref, w_ref, m_ref):
    xs = g_ref[0]
    ys = g_ref[1]
    zs = g_ref[2]
    x = jnp.clip(xs, -1.0, 1.0)
    x = (x + 1.0) / 2.0
    x = x * (_W - 1.0)
    y = jnp.clip(ys, -1.0, 1.0)
    y = (y + 1.0) / 2.0
    y = y * (_H - 1.0)
    z = jnp.clip(zs, -1.0, 1.0)
    z = (z + 1.0) / 2.0
    z = z * (_D - 1.0)
    x0 = jnp.floor(x)
    y0 = jnp.floor(y)
    z0 = jnp.floor(z)
    u = x - x0
    v = y - y0
    w = z - z0
    x1 = x0 + 1.0
    y1 = y0 + 1.0
    z1 = z0 + 1.0
    b = lax.broadcasted_iota(jnp.int32, (_B, _GW), 0)
    for j in range(_NCORN):
        dx, dy, dz = (j >> 2) & 1, (j >> 1) & 1, j & 1
        cx = x1 if dx else x0
        cy = y1 if dy else y0
        cz = z1 if dz else z0
        ix, mx = _axis_idx(cx, _W)
        iy, my = _axis_idx(cy, _H)
        iz = jnp.clip(cz.astype(jnp.int32), 0, _D - 1)
        idx_ref[j] = b * (_D * _H * _W) + iz * (_H * _W) + iy * _W + ix
        m_ref[j] = (mx & my).astype(jnp.float32)
        fx = u if dx else (1.0 - u)
        fy = v if dy else (1.0 - v)
        fz = w if dz else (1.0 - w)
        w_ref[j] = fx * fy * fz


def _compute_indices(g3):
    return pl.pallas_call(
        _idx_kernel,
        out_shape=(
            jax.ShapeDtypeStruct((_NCORN, _B, _GW), jnp.int32),
            jax.ShapeDtypeStruct((_NCORN, _B, _GW), jnp.float32),
            jax.ShapeDtypeStruct((_NCORN, _B, _GW), jnp.float32),
        ),
    )(g3)


def _sc_gather(feats_rows, idx_flat):
    mesh = plsc.VectorSubcoreMesh(core_axis_name="core",
                                  subcore_axis_name="subcore")

    @pl.kernel(out_type=jax.ShapeDtypeStruct((_NROWS, _C), jnp.float32),
               mesh=mesh)
    def gather_kernel(data_hbm, i_hbm, o_hbm):
        def body(i_vmem, o_vmem):
            pltpu.sync_copy(data_hbm.at[i_vmem.at[0]], o_vmem)

        pltpu.emit_pipeline(
            body,
            grid=(_NROWS // _GWIN,),
            in_specs=[pl.BlockSpec((1, _GWIN), lambda i: (0, i))],
            out_specs=[pl.BlockSpec((_GWIN, _C), lambda i: (i, 0))],
            core_axis_name=("core", "subcore"),
            dimension_semantics=(pltpu.PARALLEL,),
        )(i_hbm, o_hbm)

    return gather_kernel(feats_rows, idx_flat)


def _vq_kernel(x_ref, w_ref, m_ref, cb_ref, o_ref):
    rows = _NCORN * _TG
    cb = cb_ref[...]
    x3 = x_ref[...].reshape(_NCORN, _TG, _C)
    mt = m_ref[...].reshape(_NCORN, _TG, 1)
    wt = w_ref[...].reshape(_NCORN, _TG, 1)
    x2 = (x3 * mt).reshape(rows, _C)
    xx = jnp.sum(x2 * x2, axis=-1, keepdims=True)
    dot = lax.dot_general(x2, cb, (((1,), (1,)), ((), ())),
                          preferred_element_type=jnp.float32,
                          precision=lax.Precision.DEFAULT)
    cc = jnp.sum(cb * cb, axis=-1)
    d = xx - 2.0 * dot + cc
    dmin = jnp.min(d, axis=-1, keepdims=True)
    kk = lax.broadcasted_iota(jnp.int32, (rows, _K), 1)
    cand = jnp.where(d == dmin, kk, _K)
    sel = jnp.min(cand, axis=-1, keepdims=True)
    sel3 = sel.reshape(_NCORN, _TG, 1)
    kk2 = lax.broadcasted_iota(jnp.int32, (_TG, _K), 1)
    msel = jnp.where(sel3[0] == kk2, wt[0], 0.0)
    for j in range(1, _NCORN):
        msel = msel + jnp.where(sel3[j] == kk2, wt[j], 0.0)
    q = lax.dot_general(msel, cb, (((1,), (0,)), ((), ())),
                        preferred_element_type=jnp.float32,
                        precision=lax.Precision.HIGHEST)
    o_ref[...] = q.reshape(1, _TG, _C)


def _vq_blend(x4, w8, m8, codebook):
    return pl.pallas_call(
        _vq_kernel,
        grid=(_B, _GW // _TG),
        in_specs=[
            pl.BlockSpec((_NCORN, 1, _TG, _C), lambda b, t: (0, b, t, 0)),
            pl.BlockSpec((_NCORN, 1, 1, _TG), lambda b, t: (0, b, 0, t)),
            pl.BlockSpec((_NCORN, 1, 1, _TG), lambda b, t: (0, b, 0, t)),
            pl.BlockSpec((_K, _C), lambda b, t: (0, 0)),
        ],
        out_specs=pl.BlockSpec((1, _TG, _C), lambda b, t: (b, t, 0)),
        out_shape=jax.ShapeDtypeStruct((_B, _GW, _C), jnp.float32),
    )(x4, w8.reshape(_NCORN, _B, 1, _GW), m8.reshape(_NCORN, _B, 1, _GW),
      codebook)


def kernel(input_feats, sampling_grid, codebook):
    g3 = jnp.transpose(sampling_grid.reshape(_B, _GW, 3), (2, 0, 1))
    idx, w8, m8 = _compute_indices(g3)
    feats_rows = jnp.transpose(input_feats, (0, 2, 3, 4, 1))
    feats_rows = feats_rows.reshape(_B * _D * _H * _W, _C)
    xg = _sc_gather(feats_rows, idx.reshape(1, _NROWS))
    x4 = xg.reshape(_NCORN, _B, _GW, _C)
    out = _vq_blend(x4, w8, m8, codebook)
    return jnp.transpose(out, (0, 2, 1))[:, :, None, :]


# speedup vs baseline: 13.4232x; 1.2336x over previous
import jax
import jax.numpy as jnp
from jax import lax
from jax.experimental import pallas as pl
from jax.experimental.pallas import tpu as pltpu
from jax.experimental.pallas import tpu_sc as plsc

_B, _C, _D, _H, _W = 2, 256, 16, 32, 32
_GW = 2048
_K = 1024
_NCORN = 8
_NROWS = _B * _NCORN * _GW
_GWIN = 128
_TG = 256


def _axis_idx(c, n):
    half = (n - 1.0) / 2.0
    cg = (c - half) / half
    cg = jnp.clip(cg, -1.0, 1.0)
    i = ((cg + 1.0) * n - 1.0) / 2.0
    ri = jnp.round(i).astype(jnp.int32)
    valid = (ri >= 0) & (ri < n)
    return jnp.clip(ri, 0, int(n) - 1), valid


def _idx_kernel(g_ref, idx_ref, w_ref, m_ref):
    xs = g_ref[0]
    ys = g_ref[1]
    zs = g_ref[2]
    x = jnp.clip(xs, -1.0, 1.0)
    x = (x + 1.0) / 2.0
    x = x * (_W - 1.0)
    y = jnp.clip(ys, -1.0, 1.0)
    y = (y + 1.0) / 2.0
    y = y * (_H - 1.0)
    z = jnp.clip(zs, -1.0, 1.0)
    z = (z + 1.0) / 2.0
    z = z * (_D - 1.0)
    x0 = jnp.floor(x)
    y0 = jnp.floor(y)
    z0 = jnp.floor(z)
    u = x - x0
    v = y - y0
    w = z - z0
    x1 = x0 + 1.0
    y1 = y0 + 1.0
    z1 = z0 + 1.0
    b = lax.broadcasted_iota(jnp.int32, (_B, _GW), 0)
    for j in range(_NCORN):
        dx, dy, dz = (j >> 2) & 1, (j >> 1) & 1, j & 1
        cx = x1 if dx else x0
        cy = y1 if dy else y0
        cz = z1 if dz else z0
        ix, mx = _axis_idx(cx, _W)
        iy, my = _axis_idx(cy, _H)
        iz = jnp.clip(cz.astype(jnp.int32), 0, _D - 1)
        idx_ref[j] = b * (_D * _H * _W) + iz * (_H * _W) + iy * _W + ix
        m_ref[j] = (mx & my).astype(jnp.float32)
        fx = u if dx else (1.0 - u)
        fy = v if dy else (1.0 - v)
        fz = w if dz else (1.0 - w)
        w_ref[j] = fx * fy * fz


def _compute_indices(g3):
    return pl.pallas_call(
        _idx_kernel,
        out_shape=(
            jax.ShapeDtypeStruct((_NCORN, _B, _GW), jnp.int32),
            jax.ShapeDtypeStruct((_NCORN, _B, _GW), jnp.float32),
            jax.ShapeDtypeStruct((_NCORN, _B, _GW), jnp.float32),
        ),
    )(g3)


def _sc_gather(feats_rows, idx_flat):
    mesh = plsc.VectorSubcoreMesh(core_axis_name="core",
                                  subcore_axis_name="subcore")

    @pl.kernel(out_type=jax.ShapeDtypeStruct((_NROWS, _C), jnp.float32),
               mesh=mesh)
    def gather_kernel(data_hbm, i_hbm, o_hbm):
        def body(i_vmem, o_vmem):
            pltpu.sync_copy(data_hbm.at[i_vmem.at[0]], o_vmem)

        pltpu.emit_pipeline(
            body,
            grid=(_NROWS // _GWIN,),
            in_specs=[pl.BlockSpec((1, _GWIN), lambda i: (0, i))],
            out_specs=[pl.BlockSpec((_GWIN, _C), lambda i: (i, 0))],
            core_axis_name=("core", "subcore"),
            dimension_semantics=(pltpu.PARALLEL,),
        )(i_hbm, o_hbm)

    return gather_kernel(feats_rows, idx_flat)


def _vq_kernel(x_ref, w_ref, m_ref, cb_ref, o_ref):
    rows = _NCORN * _TG
    cb = cb_ref[...]
    x3 = x_ref[...].reshape(_NCORN, _TG, _C)
    mt = m_ref[...].reshape(_NCORN, _TG, 1)
    wt = w_ref[...].reshape(_NCORN, _TG, 1)
    xs = (x3 * (mt * -2.0)).reshape(rows, _C)
    xx = jnp.sum(xs * xs, axis=-1, keepdims=True) * 0.25
    dot = lax.dot_general(xs, cb, (((1,), (1,)), ((), ())),
                          preferred_element_type=jnp.float32,
                          precision=lax.Precision.DEFAULT)
    cc = jnp.sum(cb * cb, axis=-1)
    d = (xx + dot) + cc
    dmin = jnp.min(d, axis=-1, keepdims=True)
    kk = lax.broadcasted_iota(jnp.int32, (rows, _K), 1).astype(jnp.float32)
    cand = jnp.where(d == dmin, kk, float(_K))
    sel = jnp.min(cand, axis=-1, keepdims=True)
    sel3 = sel.reshape(_NCORN, _TG, 1)
    kk2 = lax.broadcasted_iota(jnp.int32, (_TG, _K), 1).astype(jnp.float32)
    msel = jnp.where(sel3[0] == kk2, wt[0], 0.0)
    for j in range(1, _NCORN):
        msel = msel + jnp.where(sel3[j] == kk2, wt[j], 0.0)
    mh = msel.astype(jnp.bfloat16)
    ml = (msel - mh.astype(jnp.float32)).astype(jnp.bfloat16)
    bh = cb.astype(jnp.bfloat16)
    bl = (cb - bh.astype(jnp.float32)).astype(jnp.bfloat16)
    dn = (((1,), (0,)), ((), ()))
    q = (lax.dot_general(mh, bh, dn, preferred_element_type=jnp.float32)
         + lax.dot_general(mh, bl, dn, preferred_element_type=jnp.float32)
         + lax.dot_general(ml, bh, dn, preferred_element_type=jnp.float32))
    o_ref[...] = q.reshape(1, _TG, _C)


def _vq_blend(x4, w8, m8, codebook):
    return pl.pallas_call(
        _vq_kernel,
        grid=(_B, _GW // _TG),
        in_specs=[
            pl.BlockSpec((_NCORN, 1, _TG, _C), lambda b, t: (0, b, t, 0)),
            pl.BlockSpec((_NCORN, 1, 1, _TG), lambda b, t: (0, b, 0, t)),
            pl.BlockSpec((_NCORN, 1, 1, _TG), lambda b, t: (0, b, 0, t)),
            pl.BlockSpec((_K, _C), lambda b, t: (0, 0)),
        ],
        out_specs=pl.BlockSpec((1, _TG, _C), lambda b, t: (b, t, 0)),
        out_shape=jax.ShapeDtypeStruct((_B, _GW, _C), jnp.float32),
    )(x4, w8.reshape(_NCORN, _B, 1, _GW), m8.reshape(_NCORN, _B, 1, _GW),
      codebook)


def kernel(input_feats, sampling_grid, codebook):
    g3 = jnp.transpose(sampling_grid.reshape(_B, _GW, 3), (2, 0, 1))
    idx, w8, m8 = _compute_indices(g3)
    feats_rows = jnp.transpose(input_feats, (0, 2, 3, 4, 1))
    feats_rows = feats_rows.reshape(_B * _D * _H * _W, _C)
    xg = _sc_gather(feats_rows, idx.reshape(1, _NROWS))
    x4 = xg.reshape(_NCORN, _B, _GW, _C)
    out = _vq_blend(x4, w8, m8, codebook)
    return jnp.transpose(out, (0, 2, 1))[:, :, None, :]
